# Initial kernel scaffold; baseline (speedup 1.0000x reference)
import jax
import jax.numpy as jnp
from jax.experimental import pallas as pl


def kernel(features, in_degree, out_degree, in_w, out_w, graph_token):
    raise NotImplementedError("write your pallas kernel here")


# speedup vs baseline: 1.3118x; 1.3118x over previous
import functools

import jax
import jax.numpy as jnp
from jax import lax
from jax.experimental import pallas as pl
from jax.experimental.pallas import tpu as pltpu
from jax.experimental.pallas import tpu_sc as plsc

_B, _N, _D = 16, 2048, 256
_ROWS = _B * _N
_NW = 32
_RPW = _ROWS // _NW
_CHUNK = 128
_NCH = _RPW // _CHUNK
_LANES = 16


@functools.partial(
    pl.kernel,
    mesh=plsc.VectorSubcoreMesh(core_axis_name="c", subcore_axis_name="s"),
    out_type=jax.ShapeDtypeStruct((_B * (_N + 1) * _D,), jnp.float32),
    scratch_types=[
        pltpu.VMEM((_RPW,), jnp.int32),
        pltpu.VMEM((_RPW,), jnp.int32),
        pltpu.VMEM((_CHUNK * _D,), jnp.float32),
        pltpu.VMEM((_CHUNK, _D), jnp.float32),
        pltpu.VMEM((_CHUNK, _D), jnp.float32),
        pltpu.VMEM((_D,), jnp.float32),
        pltpu.SemaphoreType.DMA,
        pltpu.SemaphoreType.DMA,
        pltpu.SemaphoreType.DMA,
    ],
)
def _sc_node_feature(feat_hbm, idxin_hbm, idxout_hbm, inw_hbm, outw_hbm,
                     gt_hbm, out_hbm,
                     idxin_v, idxout_v, acc_v, inr_v, outr_v, gt_v,
                     sem_f, sem_i, sem_o):
    c = lax.axis_index("c")
    s = lax.axis_index("s")
    wid = s * 2 + c
    base = wid * _RPW
    batch = wid // 2
    half = wid % 2
    out_base = (batch * (_N + 1) + 1 + half * _RPW) * _D

    pltpu.sync_copy(idxin_hbm.at[pl.ds(base, _RPW)], idxin_v)
    pltpu.sync_copy(idxout_hbm.at[pl.ds(base, _RPW)], idxout_v)

    @pl.when(half == 0)
    def _():
        pltpu.sync_copy(gt_hbm, gt_v)
        pltpu.sync_copy(gt_v, out_hbm.at[pl.ds(batch * (_N + 1) * _D, _D)])

    def chunk_body(ci, carry):
        r0 = ci * _CHUNK
        cp_f = pltpu.async_copy(
            feat_hbm.at[pl.ds((base + r0) * _D, _CHUNK * _D)], acc_v, sem_f)
        cp_i = pltpu.async_copy(inw_hbm.at[idxin_v.at[pl.ds(r0, _CHUNK)]],
                                inr_v, sem_i)
        cp_o = pltpu.async_copy(outw_hbm.at[idxout_v.at[pl.ds(r0, _CHUNK)]],
                                outr_v, sem_o)
        cp_f.wait()
        cp_i.wait()
        cp_o.wait()

        def row_body(r, carry2):
            def vec_body(j, carry3):
                sl = pl.ds(j * _LANES, _LANES)
                fsl = pl.ds(r * _D + j * _LANES, _LANES)
                acc_v[fsl] = acc_v[fsl] + inr_v[r, sl] + outr_v[r, sl]
                return carry3
            return lax.fori_loop(0, _D // _LANES, vec_body, carry2)
        lax.fori_loop(0, _CHUNK, row_body, 0)

        pltpu.sync_copy(acc_v, out_hbm.at[pl.ds(out_base + r0 * _D,
                                                _CHUNK * _D)])
        return carry
    lax.fori_loop(0, _NCH, chunk_body, 0)


def kernel(features, in_degree, out_degree, in_w, out_w, graph_token):
    feat_flat = features.reshape(_ROWS * _D)
    idx_in = in_degree.astype(jnp.int32).reshape(_ROWS)
    idx_out = out_degree.astype(jnp.int32).reshape(_ROWS)
    out = _sc_node_feature(feat_flat, idx_in, idx_out, in_w, out_w,
                           graph_token.reshape(_D))
    return out.reshape(_B, _N + 1, _D)


# speedup vs baseline: 1.6061x; 1.2244x over previous
import functools

import jax
import jax.numpy as jnp
from jax import lax
from jax.experimental import pallas as pl
from jax.experimental.pallas import tpu as pltpu
from jax.experimental.pallas import tpu_sc as plsc

_B, _N, _D = 16, 2048, 256
_ROWS = _B * _N
_NW = 32
_RPW = _ROWS // _NW
_CHUNK = 32
_NCH = _RPW // _CHUNK
_PAIRS = _NCH // 2
_LANES = 16
_GRP = _D // _LANES


@functools.partial(
    pl.kernel,
    mesh=plsc.VectorSubcoreMesh(core_axis_name="c", subcore_axis_name="s"),
    out_type=jax.ShapeDtypeStruct((_B * (_N + 1) * _D,), jnp.float32),
    scratch_types=[
        pltpu.VMEM((_RPW,), jnp.int32),
        pltpu.VMEM((_RPW,), jnp.int32),
        pltpu.VMEM((_CHUNK * _D,), jnp.float32),
        pltpu.VMEM((_CHUNK, _D), jnp.float32),
        pltpu.VMEM((_CHUNK, _D), jnp.float32),
        pltpu.VMEM((_CHUNK * _D,), jnp.float32),
        pltpu.VMEM((_CHUNK * _D,), jnp.float32),
        pltpu.VMEM((_CHUNK, _D), jnp.float32),
        pltpu.VMEM((_CHUNK, _D), jnp.float32),
        pltpu.VMEM((_CHUNK * _D,), jnp.float32),
        pltpu.VMEM((_D,), jnp.float32),
        pltpu.SemaphoreType.DMA,
        pltpu.SemaphoreType.DMA,
        pltpu.SemaphoreType.DMA,
        pltpu.SemaphoreType.DMA,
        pltpu.SemaphoreType.DMA,
        pltpu.SemaphoreType.DMA,
        pltpu.SemaphoreType.DMA,
        pltpu.SemaphoreType.DMA,
    ],
)
def _sc_node_feature(feat_hbm, idxin_hbm, idxout_hbm, inw_hbm, outw_hbm,
                     gt_hbm, out_hbm,
                     idxin_v, idxout_v,
                     feat_a, inr_a, outr_a, st_a,
                     feat_b, inr_b, outr_b, st_b,
                     gt_v,
                     sf_a, si_a, so_a, sst_a,
                     sf_b, si_b, so_b, sst_b):
    c = lax.axis_index("c")
    s = lax.axis_index("s")
    wid = s * 2 + c
    base = wid * _RPW
    batch = wid // 2
    half = wid % 2
    out_base = (batch * (_N + 1) + 1 + half * _RPW) * _D

    sets = (
        (feat_a, inr_a, outr_a, st_a, sf_a, si_a, so_a, sst_a),
        (feat_b, inr_b, outr_b, st_b, sf_b, si_b, so_b, sst_b),
    )

    pltpu.sync_copy(idxin_hbm.at[pl.ds(base, _RPW)], idxin_v)
    pltpu.sync_copy(idxout_hbm.at[pl.ds(base, _RPW)], idxout_v)

    @pl.when(half == 0)
    def _():
        pltpu.sync_copy(gt_hbm, gt_v)
        pltpu.sync_copy(gt_v, out_hbm.at[pl.ds(batch * (_N + 1) * _D, _D)])

    def start_in(ci, fb, ib, ob, sf, si, so):
        pltpu.async_copy(
            feat_hbm.at[pl.ds((base + ci * _CHUNK) * _D, _CHUNK * _D)],
            fb, sf)
        pltpu.async_copy(inw_hbm.at[idxin_v.at[pl.ds(ci * _CHUNK, _CHUNK)]],
                         ib, si)
        pltpu.async_copy(outw_hbm.at[idxout_v.at[pl.ds(ci * _CHUNK, _CHUNK)]],
                         ob, so)

    def wait_in(ci, fb, ib, ob, sf, si, so):
        pltpu.make_async_copy(
            feat_hbm.at[pl.ds((base + ci * _CHUNK) * _D, _CHUNK * _D)],
            fb, sf).wait()
        pltpu.make_async_copy(
            inw_hbm.at[idxin_v.at[pl.ds(ci * _CHUNK, _CHUNK)]],
            ib, si).wait()
        pltpu.make_async_copy(
            outw_hbm.at[idxout_v.at[pl.ds(ci * _CHUNK, _CHUNK)]],
            ob, so).wait()

    def out_slice(ci):
        return out_hbm.at[pl.ds(out_base + ci * _CHUNK * _D, _CHUNK * _D)]

    def compute(fb, ib, ob, sb):
        def row_body(r, carry):
            rb = r * _D
            for j in range(_GRP):
                sl = pl.ds(j * _LANES, _LANES)
                fsl = pl.ds(rb + j * _LANES, _LANES)
                sb[fsl] = fb[fsl] + ib[r, sl] + ob[r, sl]
            return carry
        lax.fori_loop(0, _CHUNK, row_body, 0)

    start_in(0, *sets[0][:3], *sets[0][4:7])
    start_in(1, *sets[1][:3], *sets[1][4:7])

    def pair_body(p, carry):
        for b in (0, 1):
            fb, ib, ob, sb, sf, si, so, sst = sets[b]
            ci = p * 2 + b
            wait_in(ci, fb, ib, ob, sf, si, so)

            @pl.when(p > 0)
            def _():
                pltpu.make_async_copy(sb, out_slice(ci - 2), sst).wait()

            compute(fb, ib, ob, sb)
            pltpu.async_copy(sb, out_slice(ci), sst)

            @pl.when(p < _PAIRS - 1)
            def _():
                start_in(ci + 2, fb, ib, ob, sf, si, so)
        return carry
    lax.fori_loop(0, _PAIRS, pair_body, 0)

    pltpu.make_async_copy(st_a, out_slice(_NCH - 2), sst_a).wait()
    pltpu.make_async_copy(st_b, out_slice(_NCH - 1), sst_b).wait()


def kernel(features, in_degree, out_degree, in_w, out_w, graph_token):
    feat_flat = features.reshape(_ROWS * _D)
    idx_in = in_degree.astype(jnp.int32).reshape(_ROWS)
    idx_out = out_degree.astype(jnp.int32).reshape(_ROWS)
    out = _sc_node_feature(feat_flat, idx_in, idx_out, in_w, out_w,
                           graph_token.reshape(_D))
    return out.reshape(_B, _N + 1, _D)


# speedup vs baseline: 2.6277x; 1.6361x over previous
import functools

import jax
import jax.numpy as jnp
from jax import lax
from jax.experimental import pallas as pl
from jax.experimental.pallas import tpu as pltpu
from jax.experimental.pallas import tpu_sc as plsc

_B, _N, _D = 16, 2048, 256
_ROWS = _B * _N
_NW = 32
_RPW = _ROWS // _NW
_CHUNK = 32
_NCH = _RPW // _CHUNK
_PAIRS = _NCH // 2
_LANES = 16
_GRP = _D // _LANES


@functools.partial(
    pl.kernel,
    mesh=plsc.VectorSubcoreMesh(core_axis_name="c", subcore_axis_name="s"),
    out_type=jax.ShapeDtypeStruct((_B * (_N + 1), _D), jnp.float32),
    scratch_types=[
        pltpu.VMEM((_RPW,), jnp.int32),
        pltpu.VMEM((_RPW,), jnp.int32),
        pltpu.VMEM((_NCH, _CHUNK), jnp.int32),
        pltpu.VMEM((_CHUNK, _D), jnp.float32),
        pltpu.VMEM((_CHUNK, _D), jnp.float32),
        pltpu.VMEM((_CHUNK, _D), jnp.float32),
        pltpu.VMEM((_CHUNK, _D), jnp.float32),
        pltpu.VMEM((_CHUNK, _D), jnp.float32),
        pltpu.VMEM((_CHUNK, _D), jnp.float32),
        pltpu.VMEM((_CHUNK, _D), jnp.float32),
        pltpu.VMEM((_CHUNK, _D), jnp.float32),
        pltpu.VMEM((_LANES, _D), jnp.float32),
        pltpu.SemaphoreType.DMA,
        pltpu.SemaphoreType.DMA,
        pltpu.SemaphoreType.DMA,
        pltpu.SemaphoreType.DMA,
        pltpu.SemaphoreType.DMA,
        pltpu.SemaphoreType.DMA,
        pltpu.SemaphoreType.DMA,
        pltpu.SemaphoreType.DMA,
        pltpu.SemaphoreType.DMA,
    ],
)
def _sc_node_feature(feat_hbm, idxin_hbm, idxout_hbm, inw_hbm, outw_hbm,
                     gt_hbm, out_hbm,
                     idxin_v, idxout_v, orow_v,
                     feat_a, inr_a, outr_a, st_a,
                     feat_b, inr_b, outr_b, st_b,
                     gt_v,
                     sf_a, si_a, so_a, sst_a,
                     sf_b, si_b, so_b, sst_b, sgt):
    c = lax.axis_index("c")
    s = lax.axis_index("s")
    wid = s * 2 + c
    base = wid * _RPW
    batch = wid // 2
    half = wid % 2
    out_base = batch * (_N + 1) + 1 + half * _RPW

    sets = (
        (feat_a, inr_a, outr_a, st_a, sf_a, si_a, so_a, sst_a),
        (feat_b, inr_b, outr_b, st_b, sf_b, si_b, so_b, sst_b),
    )

    pltpu.sync_copy(idxin_hbm.at[pl.ds(base, _RPW)], idxin_v)
    pltpu.sync_copy(idxout_hbm.at[pl.ds(base, _RPW)], idxout_v)

    lane = lax.iota(jnp.int32, _LANES)

    def orow_body(k, carry):
        for jj in range(_CHUNK // _LANES):
            orow_v[k, pl.ds(jj * _LANES, _LANES)] = (
                out_base + k * _CHUNK + jj * _LANES + lane)
        return carry
    lax.fori_loop(0, _NCH, orow_body, 0)

    @pl.when(wid == 0)
    def _():
        pltpu.sync_copy(gt_hbm, gt_v.at[pl.ds(0, 1)])
        for j in range(_GRP):
            sl = pl.ds(j * _LANES, _LANES)
            row = gt_v[0, sl]
            for r in range(1, _LANES):
                gt_v[r, sl] = row
        tok_rows = lane * (_N + 1)
        pltpu.async_copy(gt_v, out_hbm.at[tok_rows], sgt).wait()

    def start_in(ci, fb, ib, ob, sf, si, so):
        pltpu.async_copy(feat_hbm.at[pl.ds(base + ci * _CHUNK, _CHUNK)],
                         fb, sf)
        pltpu.async_copy(inw_hbm.at[idxin_v.at[pl.ds(ci * _CHUNK, _CHUNK)]],
                         ib, si)
        pltpu.async_copy(outw_hbm.at[idxout_v.at[pl.ds(ci * _CHUNK, _CHUNK)]],
                         ob, so)

    def wait_in(ci, fb, ib, ob, sf, si, so):
        pltpu.make_async_copy(
            feat_hbm.at[pl.ds(base + ci * _CHUNK, _CHUNK)], fb, sf).wait()
        pltpu.make_async_copy(
            inw_hbm.at[idxin_v.at[pl.ds(ci * _CHUNK, _CHUNK)]],
            ib, si).wait()
        pltpu.make_async_copy(
            outw_hbm.at[idxout_v.at[pl.ds(ci * _CHUNK, _CHUNK)]],
            ob, so).wait()

    def out_ref(ci):
        return out_hbm.at[orow_v.at[ci]]

    def compute(fb, ib, ob, sb):
        def row_body(r, carry):
            for j in range(_GRP):
                sl = pl.ds(j * _LANES, _LANES)
                sb[r, sl] = fb[r, sl] + ib[r, sl] + ob[r, sl]
            return carry
        lax.fori_loop(0, _CHUNK, row_body, 0)

    start_in(0, *sets[0][:3], *sets[0][4:7])
    start_in(1, *sets[1][:3], *sets[1][4:7])

    def pair_body(p, carry):
        for b in (0, 1):
            fb, ib, ob, sb, sf, si, so, sst = sets[b]
            ci = p * 2 + b
            wait_in(ci, fb, ib, ob, sf, si, so)

            @pl.when(p > 0)
            def _():
                pltpu.make_async_copy(sb, out_ref(ci - 2), sst).wait()

            compute(fb, ib, ob, sb)
            pltpu.async_copy(sb, out_ref(ci), sst)

            @pl.when(p < _PAIRS - 1)
            def _():
                start_in(ci + 2, fb, ib, ob, sf, si, so)
        return carry
    lax.fori_loop(0, _PAIRS, pair_body, 0)

    pltpu.make_async_copy(st_a, out_ref(_NCH - 2), sst_a).wait()
    pltpu.make_async_copy(st_b, out_ref(_NCH - 1), sst_b).wait()


def kernel(features, in_degree, out_degree, in_w, out_w, graph_token):
    feat_flat = features.reshape(_ROWS, _D)
    idx_in = in_degree.astype(jnp.int32).reshape(_ROWS)
    idx_out = out_degree.astype(jnp.int32).reshape(_ROWS)
    out = _sc_node_feature(feat_flat, idx_in, idx_out, in_w, out_w,
                           graph_token)
    return out.reshape(_B, _N + 1, _D)


# speedup vs baseline: 3.1203x; 1.1875x over previous
import functools

import jax
import jax.numpy as jnp
from jax import lax
from jax.experimental import pallas as pl
from jax.experimental.pallas import tpu as pltpu
from jax.experimental.pallas import tpu_sc as plsc

_B, _N, _D = 16, 2048, 256
_ROWS = _B * _N
_NW = 32
_RPW = _ROWS // _NW
_CHUNK = 32
_NCH = _RPW // _CHUNK
_PAIRS = _NCH // 2
_LANES = 16
_GRP = _D // _LANES


@functools.partial(
    pl.kernel,
    mesh=plsc.VectorSubcoreMesh(core_axis_name="c", subcore_axis_name="s"),
    out_type=jax.ShapeDtypeStruct((_B, _N + 1, _D), jnp.float32),
    scratch_types=[
        pltpu.VMEM((_RPW,), jnp.int32),
        pltpu.VMEM((_RPW,), jnp.int32),
        pltpu.VMEM((_NCH, _CHUNK), jnp.int32),
        pltpu.VMEM((_CHUNK, _D), jnp.float32),
        pltpu.VMEM((_CHUNK, _D), jnp.float32),
        pltpu.VMEM((_CHUNK, _D), jnp.float32),
        pltpu.VMEM((_CHUNK, _D), jnp.float32),
        pltpu.VMEM((_CHUNK, _D), jnp.float32),
        pltpu.VMEM((_CHUNK, _D), jnp.float32),
        pltpu.VMEM((_CHUNK, _D), jnp.float32),
        pltpu.VMEM((_CHUNK, _D), jnp.float32),
        pltpu.VMEM((1, _D), jnp.float32),
        pltpu.SemaphoreType.DMA,
        pltpu.SemaphoreType.DMA,
        pltpu.SemaphoreType.DMA,
        pltpu.SemaphoreType.DMA,
        pltpu.SemaphoreType.DMA,
        pltpu.SemaphoreType.DMA,
        pltpu.SemaphoreType.DMA,
        pltpu.SemaphoreType.DMA,
    ],
)
def _sc_node_feature(feat_hbm, idxin_hbm, idxout_hbm, inw_hbm, outw_hbm,
                     gt_hbm, out_hbm,
                     idxin_v, idxout_v, orow_v,
                     feat_a, inr_a, outr_a, st_a,
                     feat_b, inr_b, outr_b, st_b,
                     gt_v,
                     sf_a, si_a, so_a, sst_a,
                     sf_b, si_b, so_b, sst_b):
    c = lax.axis_index("c")
    s = lax.axis_index("s")
    wid = s * 2 + c
    base = wid * _RPW
    batch = wid // 2
    half = wid % 2
    n0 = half * _RPW
    orow0 = 1 + n0

    sets = (
        (feat_a, inr_a, outr_a, st_a, sf_a, si_a, so_a, sst_a),
        (feat_b, inr_b, outr_b, st_b, sf_b, si_b, so_b, sst_b),
    )

    pltpu.sync_copy(idxin_hbm.at[pl.ds(base, _RPW)], idxin_v)
    pltpu.sync_copy(idxout_hbm.at[pl.ds(base, _RPW)], idxout_v)

    lane = lax.iota(jnp.int32, _LANES)

    def orow_body(k, carry):
        for jj in range(_CHUNK // _LANES):
            orow_v[k, pl.ds(jj * _LANES, _LANES)] = (
                orow0 + k * _CHUNK + jj * _LANES + lane)
        return carry
    lax.fori_loop(0, _NCH, orow_body, 0)

    @pl.when(half == 0)
    def _():
        pltpu.sync_copy(gt_hbm, gt_v)
        pltpu.sync_copy(gt_v, out_hbm.at[batch].at[pl.ds(0, 1)])

    def start_in(ci, fb, ib, ob, sf, si, so):
        pltpu.async_copy(
            feat_hbm.at[batch].at[pl.ds(n0 + ci * _CHUNK, _CHUNK)], fb, sf)
        pltpu.async_copy(inw_hbm.at[idxin_v.at[pl.ds(ci * _CHUNK, _CHUNK)]],
                         ib, si)
        pltpu.async_copy(outw_hbm.at[idxout_v.at[pl.ds(ci * _CHUNK, _CHUNK)]],
                         ob, so)

    def wait_in(ci, fb, ib, ob, sf, si, so):
        pltpu.make_async_copy(
            feat_hbm.at[batch].at[pl.ds(n0 + ci * _CHUNK, _CHUNK)],
            fb, sf).wait()
        pltpu.make_async_copy(
            inw_hbm.at[idxin_v.at[pl.ds(ci * _CHUNK, _CHUNK)]],
            ib, si).wait()
        pltpu.make_async_copy(
            outw_hbm.at[idxout_v.at[pl.ds(ci * _CHUNK, _CHUNK)]],
            ob, so).wait()

    def out_ref(ci):
        return out_hbm.at[batch].at[orow_v.at[ci]]

    def compute(fb, ib, ob, sb):
        def row_body(r, carry):
            for j in range(_GRP):
                sl = pl.ds(j * _LANES, _LANES)
                sb[r, sl] = fb[r, sl] + ib[r, sl] + ob[r, sl]
            return carry
        lax.fori_loop(0, _CHUNK, row_body, 0)

    start_in(0, *sets[0][:3], *sets[0][4:7])
    start_in(1, *sets[1][:3], *sets[1][4:7])

    def pair_body(p, carry):
        for b in (0, 1):
            fb, ib, ob, sb, sf, si, so, sst = sets[b]
            ci = p * 2 + b
            wait_in(ci, fb, ib, ob, sf, si, so)

            @pl.when(p > 0)
            def _():
                pltpu.make_async_copy(sb, out_ref(ci - 2), sst).wait()

            compute(fb, ib, ob, sb)
            pltpu.async_copy(sb, out_ref(ci), sst)

            @pl.when(p < _PAIRS - 1)
            def _():
                start_in(ci + 2, fb, ib, ob, sf, si, so)
        return carry
    lax.fori_loop(0, _PAIRS, pair_body, 0)

    pltpu.make_async_copy(st_a, out_ref(_NCH - 2), sst_a).wait()
    pltpu.make_async_copy(st_b, out_ref(_NCH - 1), sst_b).wait()


def kernel(features, in_degree, out_degree, in_w, out_w, graph_token):
    idx_in = in_degree.astype(jnp.int32).reshape(_ROWS)
    idx_out = out_degree.astype(jnp.int32).reshape(_ROWS)
    return _sc_node_feature(features, idx_in, idx_out, in_w, out_w,
                            graph_token)


# speedup vs baseline: 4.4688x; 1.4322x over previous
import functools

import jax
import jax.numpy as jnp
from jax import lax
from jax.experimental import pallas as pl
from jax.experimental.pallas import tpu as pltpu
from jax.experimental.pallas import tpu_sc as plsc

_B, _N, _D = 16, 2048, 256
_ROWS = _B * _N
_NW = 32
_RPW = _ROWS // _NW
_CHUNK = 32
_NCH = _RPW // _CHUNK
_PAIRS = _NCH // 2
_LANES = 16
_GRP = _D // _LANES
_REP = 8
_TAB = _REP * 513


@functools.partial(
    pl.kernel,
    mesh=plsc.VectorSubcoreMesh(core_axis_name="c", subcore_axis_name="s"),
    out_type=jax.ShapeDtypeStruct(((_N + 1) * _B, _D), jnp.float32),
    scratch_types=[
        pltpu.VMEM((_RPW,), jnp.int32),
        pltpu.VMEM((_RPW,), jnp.int32),
        pltpu.VMEM((_NCH, _CHUNK), jnp.int32),
        pltpu.VMEM((_CHUNK, _D), jnp.float32),
        pltpu.VMEM((_CHUNK, _D), jnp.float32),
        pltpu.VMEM((_CHUNK, _D), jnp.float32),
        pltpu.VMEM((_CHUNK, _D), jnp.float32),
        pltpu.VMEM((_CHUNK, _D), jnp.float32),
        pltpu.VMEM((_CHUNK, _D), jnp.float32),
        pltpu.VMEM((_CHUNK, _D), jnp.float32),
        pltpu.VMEM((_CHUNK, _D), jnp.float32),
        pltpu.VMEM((_LANES, _D), jnp.float32),
        pltpu.SemaphoreType.DMA,
        pltpu.SemaphoreType.DMA,
        pltpu.SemaphoreType.DMA,
        pltpu.SemaphoreType.DMA,
        pltpu.SemaphoreType.DMA,
        pltpu.SemaphoreType.DMA,
        pltpu.SemaphoreType.DMA,
        pltpu.SemaphoreType.DMA,
        pltpu.SemaphoreType.DMA,
    ],
)
def _sc_node_feature(feat_hbm, idxin_hbm, idxout_hbm, inw_hbm, outw_hbm,
                     gt_hbm, out_hbm,
                     idxin_v, idxout_v, orow_v,
                     feat_a, inr_a, outr_a, st_a,
                     feat_b, inr_b, outr_b, st_b,
                     gt_v,
                     sf_a, si_a, so_a, sst_a,
                     sf_b, si_b, so_b, sst_b, sgt):
    c = lax.axis_index("c")
    s = lax.axis_index("s")
    wid = s * 2 + c
    base = wid * _RPW
    batch = wid // 2
    half = wid % 2
    n0 = half * _RPW
    orow0 = 1 + n0

    sets = (
        (feat_a, inr_a, outr_a, st_a, sf_a, si_a, so_a, sst_a),
        (feat_b, inr_b, outr_b, st_b, sf_b, si_b, so_b, sst_b),
    )

    pltpu.sync_copy(idxin_hbm.at[pl.ds(base, _RPW)], idxin_v)
    pltpu.sync_copy(idxout_hbm.at[pl.ds(base, _RPW)], idxout_v)

    rep_off = (wid % _REP) * (_TAB // _REP)

    def rep_body(i, carry):
        sl = pl.ds(i * _LANES, _LANES)
        idxin_v[sl] = idxin_v[sl] + rep_off
        idxout_v[sl] = idxout_v[sl] + rep_off
        return carry
    lax.fori_loop(0, _RPW // _LANES, rep_body, 0)

    lane = lax.iota(jnp.int32, _LANES)

    def orow_body(k, carry):
        for jj in range(_CHUNK // _LANES):
            orow_v[k, pl.ds(jj * _LANES, _LANES)] = (
                (orow0 + k * _CHUNK + jj * _LANES + lane) * _B + batch)
        return carry
    lax.fori_loop(0, _NCH, orow_body, 0)

    @pl.when(wid == 0)
    def _():
        pltpu.sync_copy(gt_hbm, gt_v.at[pl.ds(0, 1)])
        for j in range(_GRP):
            sl = pl.ds(j * _LANES, _LANES)
            row = gt_v[0, sl]
            for r in range(1, _LANES):
                gt_v[r, sl] = row
        pltpu.async_copy(gt_v, out_hbm.at[lane], sgt).wait()

    def start_in(ci, fb, ib, ob, sf, si, so):
        pltpu.async_copy(
            feat_hbm.at[batch].at[pl.ds(n0 + ci * _CHUNK, _CHUNK)], fb, sf)
        pltpu.async_copy(inw_hbm.at[idxin_v.at[pl.ds(ci * _CHUNK, _CHUNK)]],
                         ib, si)
        pltpu.async_copy(outw_hbm.at[idxout_v.at[pl.ds(ci * _CHUNK, _CHUNK)]],
                         ob, so)

    def wait_in(ci, fb, ib, ob, sf, si, so):
        pltpu.make_async_copy(
            feat_hbm.at[batch].at[pl.ds(n0 + ci * _CHUNK, _CHUNK)],
            fb, sf).wait()
        pltpu.make_async_copy(
            inw_hbm.at[idxin_v.at[pl.ds(ci * _CHUNK, _CHUNK)]],
            ib, si).wait()
        pltpu.make_async_copy(
            outw_hbm.at[idxout_v.at[pl.ds(ci * _CHUNK, _CHUNK)]],
            ob, so).wait()

    def out_ref(ci):
        return out_hbm.at[orow_v.at[ci]]

    def compute(fb, ib, ob, sb):
        def row_body(r, carry):
            for j in range(_GRP):
                sl = pl.ds(j * _LANES, _LANES)
                sb[r, sl] = fb[r, sl] + ib[r, sl] + ob[r, sl]
            return carry
        lax.fori_loop(0, _CHUNK, row_body, 0)

    start_in(0, *sets[0][:3], *sets[0][4:7])
    start_in(1, *sets[1][:3], *sets[1][4:7])

    def pair_body(p, carry):
        for b in (0, 1):
            fb, ib, ob, sb, sf, si, so, sst = sets[b]
            ci = p * 2 + b
            wait_in(ci, fb, ib, ob, sf, si, so)

            @pl.when(p > 0)
            def _():
                pltpu.make_async_copy(sb, out_ref(ci - 2), sst).wait()

            compute(fb, ib, ob, sb)
            pltpu.async_copy(sb, out_ref(ci), sst)

            @pl.when(p < _PAIRS - 1)
            def _():
                start_in(ci + 2, fb, ib, ob, sf, si, so)
        return carry
    lax.fori_loop(0, _PAIRS, pair_body, 0)

    pltpu.make_async_copy(st_a, out_ref(_NCH - 2), sst_a).wait()
    pltpu.make_async_copy(st_b, out_ref(_NCH - 1), sst_b).wait()


def kernel(features, in_degree, out_degree, in_w, out_w, graph_token):
    idx_in = in_degree.astype(jnp.int32).reshape(_ROWS)
    idx_out = out_degree.astype(jnp.int32).reshape(_ROWS)
    in_w_rep = jnp.tile(in_w, (_REP, 1))
    out_w_rep = jnp.tile(out_w, (_REP, 1))
    out = _sc_node_feature(features, idx_in, idx_out, in_w_rep, out_w_rep,
                           graph_token)
    return out.reshape(_N + 1, _B, _D).transpose(1, 0, 2)
